# baseline (speedup 1.0000x reference)
import functools

import jax
import jax.numpy as jnp
from jax import lax
from jax.experimental import pallas as pl
from jax.experimental.pallas import tpu as pltpu
from jax.experimental.pallas import tpu_sc as plsc

D_MODEL = 4096
EXPERTS = 64
TOPK = 8
BLK = 1024

NC = 2
NS = 16
LANES = 16
NW = NC * NS


def _matmul_kernel(x_ref, w_ref, b_ref, out_ref):
    out_ref[...] = (
        jnp.dot(x_ref[...], w_ref[...], preferred_element_type=jnp.float32)
        + b_ref[...]
    )


def _tc_logits(x2, W_topk, b2):
    rows = x2.shape[0]
    return pl.pallas_call(
        _matmul_kernel,
        grid=(rows // BLK,),
        in_specs=[
            pl.BlockSpec((BLK, D_MODEL), lambda i: (i, 0)),
            pl.BlockSpec((D_MODEL, EXPERTS), lambda i: (0, 0)),
            pl.BlockSpec((1, EXPERTS), lambda i: (0, 0)),
        ],
        out_specs=pl.BlockSpec((BLK, EXPERTS), lambda i: (i, 0)),
        out_shape=jax.ShapeDtypeStruct((rows, EXPERTS), jnp.float32),
        compiler_params=pltpu.CompilerParams(
            dimension_semantics=("arbitrary",),
        ),
    )(x2, W_topk, b2)


def _sc_router_body(rpw, logits_hbm, probs_hbm, idx_hbm, lg_v, pb_v, ix_v):
    wid = lax.axis_index("s") * NC + lax.axis_index("c")
    pltpu.sync_copy(logits_hbm.at[pl.ds(wid * rpw * EXPERTS, rpw * EXPERTS)], lg_v)

    lane = lax.iota(jnp.int32, LANES)
    zero16 = jnp.zeros((LANES,), jnp.float32)

    def zero_body(i, _):
        pb_v[pl.ds(i * LANES, LANES)] = zero16
        return 0

    lax.fori_loop(0, rpw * EXPERTS // LANES, zero_body, 0)

    ngroups = rpw // LANES

    def group_body(g, _):
        base = g * (LANES * EXPERTS) + lane * EXPERTS

        def expert_body(e, carry):
            tv = list(carry[:TOPK])
            ti = list(carry[TOPK:])
            c = plsc.load_gather(lg_v, [base + e])
            ci = jnp.zeros((LANES,), jnp.int32) + e
            for j in range(TOPK):
                swap = c > tv[j]
                ntv = jnp.where(swap, c, tv[j])
                nti = jnp.where(swap, ci, ti[j])
                c = jnp.where(swap, tv[j], c)
                ci = jnp.where(swap, ti[j], ci)
                tv[j] = ntv
                ti[j] = nti
            return tuple(tv) + tuple(ti)

        init = tuple(jnp.full((LANES,), -jnp.inf, jnp.float32) for _ in range(TOPK)) + tuple(
            jnp.zeros((LANES,), jnp.int32) for _ in range(TOPK)
        )
        res = lax.fori_loop(0, EXPERTS, expert_body, init)
        tv = res[:TOPK]
        ti = res[TOPK:]

        es = [jnp.exp(t - tv[0]) for t in tv]
        z = es[0]
        for j in range(1, TOPK):
            z = z + es[j]
        inv = 1.0 / z

        idx_base = g * (LANES * TOPK) + lane * TOPK
        for j in range(TOPK):
            plsc.store_scatter(pb_v, [base + ti[j]], es[j] * inv)
            plsc.store_scatter(ix_v, [idx_base + j], ti[j])
        return 0

    lax.fori_loop(0, ngroups, group_body, 0)

    pltpu.sync_copy(pb_v, probs_hbm.at[pl.ds(wid * rpw * EXPERTS, rpw * EXPERTS)])
    pltpu.sync_copy(ix_v, idx_hbm.at[pl.ds(wid * rpw * TOPK, rpw * TOPK)])


def _sc_router(logits_flat, rows):
    rpw = rows // NW
    mesh = plsc.VectorSubcoreMesh(
        core_axis_name="c", subcore_axis_name="s", num_cores=NC, num_subcores=NS
    )
    fn = pl.kernel(
        functools.partial(_sc_router_body, rpw),
        mesh=mesh,
        out_type=[
            jax.ShapeDtypeStruct((rows * EXPERTS,), jnp.float32),
            jax.ShapeDtypeStruct((rows * TOPK,), jnp.int32),
        ],
        scratch_types=[
            pltpu.VMEM((rpw * EXPERTS,), jnp.float32),
            pltpu.VMEM((rpw * EXPERTS,), jnp.float32),
            pltpu.VMEM((rpw * TOPK,), jnp.int32),
        ],
        compiler_params=pltpu.CompilerParams(needs_layout_passes=False),
    )
    return fn(logits_flat)


CHUNKS = 4


@jax.jit
def kernel(x, W_topk, b_topk, W_noisy, b_noisy):
    del W_noisy, b_noisy
    B, S, D = x.shape
    rows = B * S
    x2 = x.reshape(rows, D)
    b2 = b_topk.reshape(1, EXPERTS)

    rc = rows // CHUNKS
    probs_parts = []
    idx_parts = []
    for ci in range(CHUNKS):
        lg = _tc_logits(lax.slice(x2, (ci * rc, 0), ((ci + 1) * rc, D)), W_topk, b2)
        p, i = _sc_router(lg.reshape(rc * EXPERTS), rc)
        probs_parts.append(p.reshape(rc, EXPERTS))
        idx_parts.append(i.reshape(rc, TOPK))
    probs = jnp.concatenate(probs_parts, axis=0)
    idx = jnp.concatenate(idx_parts, axis=0)
    return (
        probs.reshape(B, S, EXPERTS),
        idx.reshape(B, S, TOPK),
    )


# speedup vs baseline: 1.8257x; 1.8257x over previous
import functools

import jax
import jax.numpy as jnp
from jax import lax
from jax.experimental import pallas as pl
from jax.experimental.pallas import tpu as pltpu
from jax.experimental.pallas import tpu_sc as plsc

D_MODEL = 4096
EXPERTS = 64
TOPK = 8
BLK = 1024

NC = 2
NS = 16
LANES = 16
NW = NC * NS


def _matmul_kernel(x_ref, w_ref, b_ref, out_ref):
    out_ref[...] = (
        jnp.dot(x_ref[...], w_ref[...], preferred_element_type=jnp.float32)
        + b_ref[...]
    )


def _tc_logits(x2, W_topk, b2):
    rows = x2.shape[0]
    return pl.pallas_call(
        _matmul_kernel,
        grid=(rows // BLK,),
        in_specs=[
            pl.BlockSpec((BLK, D_MODEL), lambda i: (i, 0)),
            pl.BlockSpec((D_MODEL, EXPERTS), lambda i: (0, 0)),
            pl.BlockSpec((1, EXPERTS), lambda i: (0, 0)),
        ],
        out_specs=pl.BlockSpec((BLK, EXPERTS), lambda i: (i, 0)),
        out_shape=jax.ShapeDtypeStruct((rows, EXPERTS), jnp.float32),
        compiler_params=pltpu.CompilerParams(
            dimension_semantics=("arbitrary",),
        ),
    )(x2, W_topk, b2)


def _sc_router_body(rpw, logits_hbm, probs_hbm, idx_hbm, lg_v, pb_v, ix_v):
    wid = lax.axis_index("s") * NC + lax.axis_index("c")
    pltpu.sync_copy(logits_hbm.at[pl.ds(wid * rpw * EXPERTS, rpw * EXPERTS)], lg_v)

    lane = lax.iota(jnp.int32, LANES)
    zero16 = jnp.zeros((LANES,), jnp.float32)

    def zero_body(i, _):
        pb_v[pl.ds(i * LANES, LANES)] = zero16
        return 0

    lax.fori_loop(0, rpw * EXPERTS // LANES, zero_body, 0)

    ngroups = rpw // LANES

    def group_body(g, _):
        base = g * (LANES * EXPERTS) + lane * EXPERTS

        def expert_body(e, carry):
            tv = list(carry[:TOPK])
            ti = list(carry[TOPK:])
            c = plsc.load_gather(lg_v, [base + e])
            ci = jnp.zeros((LANES,), jnp.int32) + e
            for j in range(TOPK):
                swap = c > tv[j]
                ntv = jnp.where(swap, c, tv[j])
                nti = jnp.where(swap, ci, ti[j])
                c = jnp.where(swap, tv[j], c)
                ci = jnp.where(swap, ti[j], ci)
                tv[j] = ntv
                ti[j] = nti
            return tuple(tv) + tuple(ti)

        init = tuple(jnp.full((LANES,), -jnp.inf, jnp.float32) for _ in range(TOPK)) + tuple(
            jnp.zeros((LANES,), jnp.int32) for _ in range(TOPK)
        )
        res = lax.fori_loop(0, EXPERTS, expert_body, init)
        tv = res[:TOPK]
        ti = res[TOPK:]

        es = [jnp.exp(t - tv[0]) for t in tv]
        z = es[0]
        for j in range(1, TOPK):
            z = z + es[j]
        inv = 1.0 / z

        idx_base = g * (LANES * TOPK) + lane * TOPK
        for j in range(TOPK):
            plsc.store_scatter(pb_v, [base + ti[j]], es[j] * inv)
            plsc.store_scatter(ix_v, [idx_base + j], ti[j])
        return 0

    lax.fori_loop(0, ngroups, group_body, 0)

    pltpu.sync_copy(pb_v, probs_hbm.at[pl.ds(wid * rpw * EXPERTS, rpw * EXPERTS)])
    pltpu.sync_copy(ix_v, idx_hbm.at[pl.ds(wid * rpw * TOPK, rpw * TOPK)])


def _sc_router(logits_flat, rows):
    rpw = rows // NW
    mesh = plsc.VectorSubcoreMesh(
        core_axis_name="c", subcore_axis_name="s", num_cores=NC, num_subcores=NS
    )
    fn = pl.kernel(
        functools.partial(_sc_router_body, rpw),
        mesh=mesh,
        out_type=[
            jax.ShapeDtypeStruct((rows * EXPERTS,), jnp.float32),
            jax.ShapeDtypeStruct((rows * TOPK,), jnp.int32),
        ],
        scratch_types=[
            pltpu.VMEM((rpw * EXPERTS,), jnp.float32),
            pltpu.VMEM((rpw * EXPERTS,), jnp.float32),
            pltpu.VMEM((rpw * TOPK,), jnp.int32),
        ],
        compiler_params=pltpu.CompilerParams(needs_layout_passes=False),
    )
    return fn(logits_flat)


CHUNKS = 1


@jax.jit
def kernel(x, W_topk, b_topk, W_noisy, b_noisy):
    del W_noisy, b_noisy
    B, S, D = x.shape
    rows = B * S
    x2 = x.reshape(rows, D)
    b2 = b_topk.reshape(1, EXPERTS)

    rc = rows // CHUNKS
    probs_parts = []
    idx_parts = []
    for ci in range(CHUNKS):
        lg = _tc_logits(lax.slice(x2, (ci * rc, 0), ((ci + 1) * rc, D)), W_topk, b2)
        p, i = _sc_router(lg.reshape(rc * EXPERTS), rc)
        probs_parts.append(p.reshape(rc, EXPERTS))
        idx_parts.append(i.reshape(rc, TOPK))
    probs = jnp.concatenate(probs_parts, axis=0)
    idx = jnp.concatenate(idx_parts, axis=0)
    return (
        probs.reshape(B, S, EXPERTS),
        idx.reshape(B, S, TOPK),
    )
